# baseline (speedup 1.0000x reference)
import functools

import jax
import jax.numpy as jnp
from jax import lax
from jax.experimental import pallas as pl
from jax.experimental.pallas import tpu as pltpu
from jax.experimental.pallas import tpu_sc as plsc

RAD = 0.02
THRESH = 2.0 * RAD
T2 = THRESH * THRESH
WINDOW = 0.159
NP = 2048
N = 2000
B = 8
SLABS = 4
RPW = N // SLABS
L = 16


def _iota16():
    return lax.broadcasted_iota(jnp.int32, (L,), 0)


def _bf16r(v):
    u = lax.bitcast_convert_type(v, jnp.int32)
    rb = jnp.bitwise_and(lax.shift_right_logical(u, 16), 1)
    u = jnp.bitwise_and(u + 0x7FFF + rb, jnp.int32(-65536))
    return lax.bitcast_convert_type(u, jnp.float32)


HBUF = 14336
FLUSH_AT = HBUF - 5 * NP


def _sc_body(pos_hbm, cnt_hbm, loss_hbm,
             x_ref, y_ref, z_ref, bx_ref, by_ref, bz_ref, sq_ref,
             trips_ref, hbuf_ref, cacc_ref, lacc_ref, sem):
    c = lax.axis_index("c")
    s = lax.axis_index("s")
    wid = s * 2 + c
    b = wid // SLABS
    slab = wid % SLABS
    r0 = slab * RPW

    base = b * (3 * NP)
    pltpu.sync_copy(pos_hbm.at[pl.ds(base, NP)], x_ref)
    pltpu.sync_copy(pos_hbm.at[pl.ds(base + NP, NP)], y_ref)
    pltpu.sync_copy(pos_hbm.at[pl.ds(base + 2 * NP, NP)], z_ref)

    def pre(k, _):
        sl = pl.ds(k * L, L)
        xv = x_ref[sl]
        yv = y_ref[sl]
        zv = z_ref[sl]
        bxv = _bf16r(xv)
        byv = _bf16r(yv)
        bzv = _bf16r(zv)
        sqv = xv * xv + yv * yv + zv * zv
        bx_ref[sl] = bxv
        by_ref[sl] = byv
        bz_ref[sl] = bzv
        sq_ref[sl] = sqv
        return 0

    lax.fori_loop(0, NP // L, pre, 0)

    def ends(k, _):
        i_v = r0 + k * L + _iota16()
        tgt = plsc.load_gather(x_ref, [i_v]) + WINDOW
        e = jnp.zeros((L,), jnp.int32)
        for sh in (1024, 512, 256, 128, 64, 32, 16, 8, 4, 2, 1):
            probe = e + (sh - 1)
            below = plsc.load_gather(x_ref, [probe]) < tgt
            e = e + jnp.where(below, sh, 0)
        trips = lax.shift_right_logical(e - i_v + (L - 2), 4)
        trips_ref[pl.ds(r0 + k * L, L)] = trips
        return 0

    lax.fori_loop(0, RPW // L + 1, ends, 0)

    def drain(n, loss_v):
        dtrips = lax.shift_right_logical(n + (L - 1), 4)

        def dchunk(k, lv):
            v = hbuf_ref[pl.ds(k * L, L)]
            valid = (k * L + _iota16()) < n
            a = jnp.maximum(v, 0.0) + 1e-12
            u = lax.bitcast_convert_type(a, jnp.int32)
            u = 0x5F3759DF - lax.shift_right_logical(u, 1)
            r = lax.bitcast_convert_type(u, jnp.float32)
            ha = 0.5 * a
            r = r * (1.5 - ha * r * r)
            r = r * (1.5 - ha * r * r)
            r = r * (1.5 - ha * r * r)
            pen = THRESH - a * r
            return lv + jnp.where(valid, pen * pen, 0.0)

        return lax.fori_loop(0, dtrips, dchunk, loss_v)

    def rowquad(g, carry):
        ptr, flushed, loss_v = carry
        i = r0 + 5 * g
        rows = []
        for m in range(5):
            im = i + m
            rows.append((
                jnp.full((L,), bx_ref[pl.ds(im, L)][0] * 2.0, jnp.float32),
                jnp.full((L,), by_ref[pl.ds(im, L)][0] * 2.0, jnp.float32),
                jnp.full((L,), bz_ref[pl.ds(im, L)][0] * 2.0, jnp.float32),
                jnp.full((L,), sq_ref[pl.ds(im, L)][0], jnp.float32),
                jnp.full((L,), im, jnp.int32),
                trips_ref[pl.ds(im, L)][0],
            ))

        base = i + 1
        lim = i + 1 + rows[0][5] * L
        for m in range(1, 5):
            lim = jnp.maximum(lim, i + m + 1 + rows[m][5] * L)
        gtrips = lax.shift_right_logical(lim - base + (L - 1), 4)

        @plsc.parallel_loop(0, gtrips, unroll=2, carry=ptr)
        def chunk(k, ptr):
            idx = (base + k * L) + _iota16()
            bxj = plsc.load_gather(bx_ref, [idx])
            byj = plsc.load_gather(by_ref, [idx])
            bzj = plsc.load_gather(bz_ref, [idx])
            sqj = plsc.load_gather(sq_ref, [idx])
            for m in range(5):
                bxm, bym, bzm, sqm, im_b, _ = rows[m]
                d2m = (sqm + sqj) - (bxm * bxj + bym * byj + bzm * bzj)
                hitm = d2m < T2
                if m > 0:
                    hitm = hitm & (idx > im_b)
                plsc.store_compressed(hbuf_ref.at[pl.ds(ptr, L)], d2m,
                                      mask=hitm)
                ptr = ptr + plsc.all_reduce_population_count(hitm)[0]
            return ptr

        ptr = chunk

        def flush(args):
            ptr, flushed, loss_v = args
            return (jnp.int32(0), flushed + ptr, drain(ptr, loss_v))

        return lax.cond(ptr >= FLUSH_AT, flush, lambda a: a,
                        (ptr, flushed, loss_v))

    ptr, flushed, loss_v = lax.fori_loop(
        0, RPW // 5, rowquad,
        (jnp.int32(0), jnp.int32(0), jnp.zeros((L,), jnp.float32)))

    loss_v = drain(ptr, loss_v)
    total = flushed + ptr
    cacc_ref[...] = jnp.where(_iota16() == 0, total, 0)
    lacc_ref[...] = loss_v
    pltpu.sync_copy(cacc_ref, cnt_hbm.at[pl.ds(wid * L, L)])
    pltpu.sync_copy(lacc_ref, loss_hbm.at[pl.ds(wid * L, L)])


@jax.jit
def kernel(pos):
    x = pos[:, :, 0]
    y = pos[:, :, 1]
    z = pos[:, :, 2]
    xs, ys, zs = lax.sort((x, y, z), dimension=1, num_keys=1)
    padv = jnp.exp2(7.0 + jnp.arange(NP - N, dtype=pos.dtype))
    padm = jnp.broadcast_to(padv, (B, NP - N))
    xs = jnp.concatenate([xs, padm], axis=1)
    ys = jnp.concatenate([ys, padm], axis=1)
    zs = jnp.concatenate([zs, padm], axis=1)
    pos_s = jnp.stack([xs, ys, zs], axis=1).reshape(B * 3 * NP)

    mesh = plsc.VectorSubcoreMesh(core_axis_name="c", subcore_axis_name="s")
    f = functools.partial(
        pl.kernel, _sc_body, mesh=mesh,
        compiler_params=pltpu.CompilerParams(needs_layout_passes=False),
        out_type=[
            jax.ShapeDtypeStruct((32 * L,), jnp.int32),
            jax.ShapeDtypeStruct((32 * L,), jnp.float32),
        ],
        scratch_types=[
            pltpu.VMEM((NP,), jnp.float32),
            pltpu.VMEM((NP,), jnp.float32),
            pltpu.VMEM((NP,), jnp.float32),
            pltpu.VMEM((NP,), jnp.float32),
            pltpu.VMEM((NP,), jnp.float32),
            pltpu.VMEM((NP,), jnp.float32),
            pltpu.VMEM((NP,), jnp.float32),
            pltpu.VMEM((NP,), jnp.int32),
            pltpu.VMEM((HBUF,), jnp.float32),
            pltpu.VMEM((L,), jnp.int32),
            pltpu.VMEM((L,), jnp.float32),
            pltpu.SemaphoreType.DMA,
        ],
    )()
    cnt, loss = f(pos_s)
    return (jnp.sum(cnt).astype(jnp.int32), jnp.sum(loss))
